# Initial kernel scaffold; baseline (speedup 1.0000x reference)
import functools

import jax
import jax.numpy as jnp
from jax import lax
from jax.experimental import pallas as pl
from jax.experimental.pallas import tpu as pltpu
from jax.experimental.pallas import tpu_sc as plsc

_V = 1000
_N = 16 * 2048
_NC, _NS, _L = 2, 16, 16
_NW = _NC * _NS
_RPW = _N // _NW
_CH = 64
_NCH = _RPW // _CH


def _rowlse_kernel(c_ref, lse_ref):
    x = c_ref[...]
    m = jnp.max(x, axis=1)
    lse_ref[...] = jnp.log(jnp.sum(jnp.exp(x - m[:, None]), axis=1)) + m


def _sc_body(c_hbm, idx_hbm, tgt_hbm, lse_hbm, out_hbm, part_hbm,
             idx_v, tgt_v, lse_v, rows_v, acc_v, sem):
    wid = lax.axis_index("s") * _NC + lax.axis_index("c")
    base = wid * _RPW
    pltpu.sync_copy(idx_hbm.at[pl.ds(base, _RPW)], idx_v)
    pltpu.sync_copy(tgt_hbm.at[pl.ds(base, _RPW)], tgt_v)
    pltpu.sync_copy(lse_hbm, lse_v)
    acc = jnp.zeros((_L,), jnp.float32)
    for c in range(_NCH):
        pltpu.async_copy(
            c_hbm.at[idx_v.at[pl.ds(c * _CH, _CH)]], rows_v, sem).wait()
        for g in range(_CH // _L):
            off = c * _CH + g * _L
            rvec = lax.iota(jnp.int32, _L) + g * _L
            cvec = tgt_v[pl.ds(off, _L)]
            ivec = idx_v[pl.ds(off, _L)]
            picked = plsc.load_gather(rows_v, [rvec, cvec])
            lse_g = plsc.load_gather(lse_v, [ivec])
            acc = acc + lse_g - picked
        pltpu.sync_copy(rows_v, out_hbm.at[pl.ds(base + c * _CH, _CH)])
    acc_v[...] = acc
    pltpu.sync_copy(acc_v, part_hbm.at[wid])


_sc_call = functools.partial(
    pl.kernel,
    mesh=plsc.VectorSubcoreMesh(core_axis_name="c", subcore_axis_name="s"),
    out_type=[
        jax.ShapeDtypeStruct((_N, _V), jnp.float32),
        jax.ShapeDtypeStruct((_NW, _L), jnp.float32),
    ],
    scratch_types=[
        pltpu.VMEM((_RPW,), jnp.int32),
        pltpu.VMEM((_RPW,), jnp.int32),
        pltpu.VMEM((_V,), jnp.float32),
        pltpu.VMEM((_CH, _V), jnp.float32),
        pltpu.VMEM((_L,), jnp.float32),
        pltpu.SemaphoreType.DMA,
    ],
)(_sc_body)


def kernel(input, targets, C):
    inp_f = input.reshape(-1).astype(jnp.int32)
    tgt_f = targets.reshape(-1).astype(jnp.int32)
    rowlse = pl.pallas_call(
        _rowlse_kernel,
        out_shape=jax.ShapeDtypeStruct((_V,), jnp.float32),
    )(C)
    logits, part = _sc_call(C, inp_f, tgt_f, rowlse)
    loss = jnp.sum(part) / jnp.float32(_N)
    return (logits, loss)


# speedup vs baseline: 1.1244x; 1.1244x over previous
import functools

import jax
import jax.numpy as jnp
from jax import lax
from jax.experimental import pallas as pl
from jax.experimental.pallas import tpu as pltpu
from jax.experimental.pallas import tpu_sc as plsc

_V = 1000
_N = 16 * 2048
_NC, _NS, _L = 2, 16, 16
_NW = _NC * _NS
_RPW = _N // _NW
_CH = 64
_NCH = _RPW // _CH


def _rowlse_kernel(c_ref, lse_ref):
    x = c_ref[...]
    m = jnp.max(x, axis=1)
    lse_ref[...] = jnp.log(jnp.sum(jnp.exp(x - m[:, None]), axis=1)) + m


def _sc_body(c_hbm, idx_hbm, tgt_hbm, lse_hbm, out_hbm, part_hbm,
             idx_v, tgt_v, lse_v, rows_v, acc_v, sem):
    wid = lax.axis_index("s") * _NC + lax.axis_index("c")
    base = wid * _RPW
    pltpu.sync_copy(idx_hbm.at[pl.ds(base, _RPW)], idx_v)
    pltpu.sync_copy(tgt_hbm.at[pl.ds(base, _RPW)], tgt_v)
    pltpu.sync_copy(lse_hbm, lse_v)
    acc = jnp.zeros((_L,), jnp.float32)
    for c in range(_NCH):
        pltpu.async_copy(
            c_hbm.at[idx_v.at[pl.ds(c * _CH, _CH)]], rows_v, sem).wait()
        for g in range(_CH // _L):
            off = c * _CH + g * _L
            rvec = lax.iota(jnp.int32, _L) + g * _L
            cvec = tgt_v[pl.ds(off, _L)]
            ivec = idx_v[pl.ds(off, _L)]
            picked = plsc.load_gather(rows_v, [rvec, cvec])
            lse_g = plsc.load_gather(lse_v, [ivec])
            acc = acc + lse_g - picked
        pltpu.sync_copy(rows_v, out_hbm.at[pl.ds(base + c * _CH, _CH)])
    acc_v[...] = acc
    pltpu.sync_copy(acc_v, part_hbm.at[wid])


_sc_call = functools.partial(
    pl.kernel,
    mesh=plsc.VectorSubcoreMesh(core_axis_name="c", subcore_axis_name="s"),
    compiler_params=pltpu.CompilerParams(
        use_tc_tiling_on_sc=False, needs_layout_passes=False),
    out_type=[
        jax.ShapeDtypeStruct((_N, _V), jnp.float32),
        jax.ShapeDtypeStruct((_NW, _L), jnp.float32),
    ],
    scratch_types=[
        pltpu.VMEM((_RPW,), jnp.int32),
        pltpu.VMEM((_RPW,), jnp.int32),
        pltpu.VMEM((_V,), jnp.float32),
        pltpu.VMEM((_CH, _V), jnp.float32),
        pltpu.VMEM((_L,), jnp.float32),
        pltpu.SemaphoreType.DMA,
    ],
)(_sc_body)


def kernel(input, targets, C):
    inp_f = input.reshape(-1).astype(jnp.int32)
    tgt_f = targets.reshape(-1).astype(jnp.int32)
    rowlse = pl.pallas_call(
        _rowlse_kernel,
        out_shape=jax.ShapeDtypeStruct((_V,), jnp.float32),
    )(C)
    logits, part = _sc_call(C, inp_f, tgt_f, rowlse)
    loss = jnp.sum(part) / jnp.float32(_N)
    return (logits, loss)
